# baseline (speedup 1.0000x reference)
import numpy as np

import jax
import jax.numpy as jnp
from jax.experimental import pallas as pl
from jax.experimental.pallas import tpu as pltpu

_TAU = 1.0


def _np_threefry2x32(k0, k1, x0, x1):
    u32 = np.uint32

    def rotl(x, r):
        return u32((int(x) << r | int(x) >> (32 - r)) & 0xFFFFFFFF)

    ks2 = u32(int(k0) ^ int(k1) ^ 0x1BD11BDA)
    x0 = u32((int(x0) + int(k0)) & 0xFFFFFFFF)
    x1 = u32((int(x1) + int(k1)) & 0xFFFFFFFF)
    ks = (k1, ks2, k0, k1, ks2, k0)
    rots = ((13, 15, 26, 6), (17, 29, 16, 24))
    for j in range(5):
        for r in rots[j % 2]:
            x0 = u32((int(x0) + int(x1)) & 0xFFFFFFFF)
            x1 = u32(int(rotl(x1, r)) ^ int(x0))
        x0 = u32((int(x0) + int(ks[j])) & 0xFFFFFFFF)
        x1 = u32((int(x1) + int(ks[j + 1]) + j + 1) & 0xFFFFFFFF)
    return x0, x1


def _fold_in_key42(i):
    return _np_threefry2x32(np.uint32(0), np.uint32(42),
                            np.uint32(0), np.uint32(i))


_KEYS = np.stack([np.stack(_fold_in_key42(0)),
                  np.stack(_fold_in_key42(1))]).view(np.int32)


def _threefry_loglog(k0, k1, cnt):
    u32 = np.uint32

    def rotl(x, r):
        return (x << u32(r)) | (x >> u32(32 - r))

    def rounds(x0, x1, rots):
        for r in rots:
            x0 = x0 + x1
            x1 = rotl(x1, r)
            x1 = x0 ^ x1
        return x0, x1

    ks2 = k0 ^ k1 ^ u32(0x1BD11BDA)
    ra = (13, 15, 26, 6)
    rb = (17, 29, 16, 24)
    x1 = cnt + k1
    x0 = x1 + k0
    x1 = rotl(x1, ra[0]) ^ x0
    x0, x1 = rounds(x0, x1, ra[1:])
    x0, x1 = x0 + k1, x1 + (ks2 + u32(1))
    x0, x1 = rounds(x0, x1, rb)
    x0, x1 = x0 + ks2, x1 + (k0 + u32(2))
    x0, x1 = rounds(x0, x1, ra)
    x0, x1 = x0 + k0, x1 + (k1 + u32(3))
    x0, x1 = rounds(x0, x1, rb)
    x0, x1 = x0 + k1, x1 + (ks2 + u32(4))
    x0, x1 = rounds(x0, x1, ra)
    x0, x1 = x0 + ks2, x1 + (k0 + u32(5))
    bits = x0 ^ x1

    float_bits = (bits >> u32(9)) | u32(0x3F800000)
    f = jax.lax.bitcast_convert_type(float_bits, jnp.float32)
    f = f - jnp.float32(1.0)
    mn = jnp.float32(1e-9)
    u = jnp.maximum(mn, f + mn)
    return jnp.log(-jnp.log(u))


def _make_body(blk, seq_len, vocab, edim):

    def _vq_body(keys_ref, z_ref, w0_ref, b0_ref, e0_ref,
                 w1_ref, b1_ref, e1_ref, out_ref, loss_ref):
        @pl.when(pl.program_id(0) == 0)
        def _init():
            loss_ref[0, 0] = jnp.float32(0.0)

        i = pl.program_id(0)
        t0 = i * blk
        b = t0 // seq_len
        l0 = t0 % seq_len
        base = (b * (vocab * seq_len) + l0).astype(jnp.uint32)
        row = jax.lax.broadcasted_iota(jnp.uint32, (blk, vocab), 0)
        col = jax.lax.broadcasted_iota(jnp.uint32, (blk, vocab), 1)
        cnt = base + row + col * np.uint32(seq_len)

        z = z_ref[...]
        log_v = jnp.log(jnp.float32(vocab))
        acc = jnp.float32(0.0)
        for idx, (w_ref, b_ref, e_ref) in enumerate(
                ((w0_ref, b0_ref, e0_ref), (w1_ref, b1_ref, e1_ref))):
            logits = jnp.dot(z, w_ref[...],
                             preferred_element_type=jnp.float32)
            logits = logits + b_ref[...]
            ll = _threefry_loglog(keys_ref[idx, 0].astype(jnp.uint32),
                                  keys_ref[idx, 1].astype(jnp.uint32), cnt)
            y = logits - ll
            if _TAU != 1.0:
                y = y * (1.0 / _TAU)
            y = y - jnp.max(y, axis=1, keepdims=True)
            ey = jnp.exp(y)
            soft = ey * (1.0 / jnp.sum(ey, axis=1, keepdims=True))
            out_ref[:, idx * edim:(idx + 1) * edim] = jnp.dot(
                soft.astype(jnp.bfloat16), e_ref[...],
                preferred_element_type=jnp.float32)
            x = logits - jnp.max(logits, axis=1, keepdims=True)
            ex = jnp.exp(x)
            zden = jnp.sum(ex, axis=1, keepdims=True)
            s1 = jnp.sum(ex * x, axis=1, keepdims=True)
            acc = acc + jnp.sum(s1 / zden + (log_v - jnp.log(zden)))
        loss_ref[0, 0] += acc

    return _vq_body


def kernel(seq, proj_w0, proj_b0, embed0, proj_w1, proj_b1, embed1):
    b, l, c = seq.shape
    v = proj_w0.shape[0]
    d = embed0.shape[1]
    tok = b * l

    z = seq.reshape(tok, c).astype(jnp.bfloat16)

    blk = 1024
    while l % blk:
        blk //= 2
    grid = tok // blk
    out, loss = pl.pallas_call(
        _make_body(blk, l, v, d),
        grid=(grid,),
        in_specs=[
            pl.BlockSpec(memory_space=pltpu.SMEM),
            pl.BlockSpec((blk, c), lambda i: (i, 0)),
            pl.BlockSpec((c, v), lambda i: (0, 0)),
            pl.BlockSpec((1, v), lambda i: (0, 0)),
            pl.BlockSpec((v, d), lambda i: (0, 0)),
            pl.BlockSpec((c, v), lambda i: (0, 0)),
            pl.BlockSpec((1, v), lambda i: (0, 0)),
            pl.BlockSpec((v, d), lambda i: (0, 0)),
        ],
        out_specs=[
            pl.BlockSpec((blk, 2 * d), lambda i: (i, 0)),
            pl.BlockSpec((1, 1), lambda i: (0, 0),
                         memory_space=pltpu.SMEM),
        ],
        out_shape=[
            jax.ShapeDtypeStruct((tok, 2 * d), jnp.float32),
            jax.ShapeDtypeStruct((1, 1), jnp.float32),
        ],
    )(jnp.asarray(_KEYS), z,
      proj_w0.T.astype(jnp.bfloat16), proj_b0.reshape(1, v),
      embed0.astype(jnp.bfloat16),
      proj_w1.T.astype(jnp.bfloat16), proj_b1.reshape(1, v),
      embed1.astype(jnp.bfloat16))
    return out.reshape(b, l, 2 * d), loss[0, 0] / tok


# speedup vs baseline: 1.2568x; 1.2568x over previous
import numpy as np

import jax
import jax.numpy as jnp
from jax.experimental import pallas as pl
from jax.experimental.pallas import tpu as pltpu

_TAU = 1.0


def _np_threefry2x32(k0, k1, x0, x1):
    u32 = np.uint32

    def rotl(x, r):
        return u32((int(x) << r | int(x) >> (32 - r)) & 0xFFFFFFFF)

    ks2 = u32(int(k0) ^ int(k1) ^ 0x1BD11BDA)
    x0 = u32((int(x0) + int(k0)) & 0xFFFFFFFF)
    x1 = u32((int(x1) + int(k1)) & 0xFFFFFFFF)
    ks = (k1, ks2, k0, k1, ks2, k0)
    rots = ((13, 15, 26, 6), (17, 29, 16, 24))
    for j in range(5):
        for r in rots[j % 2]:
            x0 = u32((int(x0) + int(x1)) & 0xFFFFFFFF)
            x1 = u32(int(rotl(x1, r)) ^ int(x0))
        x0 = u32((int(x0) + int(ks[j])) & 0xFFFFFFFF)
        x1 = u32((int(x1) + int(ks[j + 1]) + j + 1) & 0xFFFFFFFF)
    return x0, x1


def _fold_in_key42(i):
    return _np_threefry2x32(np.uint32(0), np.uint32(42),
                            np.uint32(0), np.uint32(i))


_KEYS = np.stack([np.stack(_fold_in_key42(0)),
                  np.stack(_fold_in_key42(1))]).view(np.int32)


def _threefry_loglog(k0, k1, cnt):
    u32 = np.uint32

    def rotl(x, r):
        return (x << u32(r)) | (x >> u32(32 - r))

    def rounds(x0, x1, rots):
        for r in rots:
            x0 = x0 + x1
            x1 = rotl(x1, r)
            x1 = x0 ^ x1
        return x0, x1

    ks2 = k0 ^ k1 ^ u32(0x1BD11BDA)
    ra = (13, 15, 26, 6)
    rb = (17, 29, 16, 24)
    x1 = cnt + k1
    x0 = x1 + k0
    x1 = rotl(x1, ra[0]) ^ x0
    x0, x1 = rounds(x0, x1, ra[1:])
    x0, x1 = x0 + k1, x1 + (ks2 + u32(1))
    x0, x1 = rounds(x0, x1, rb)
    x0, x1 = x0 + ks2, x1 + (k0 + u32(2))
    x0, x1 = rounds(x0, x1, ra)
    x0, x1 = x0 + k0, x1 + (k1 + u32(3))
    x0, x1 = rounds(x0, x1, rb)
    x0, x1 = x0 + k1, x1 + (ks2 + u32(4))
    x0, x1 = rounds(x0, x1, ra)
    x0, x1 = x0 + ks2, x1 + (k0 + u32(5))
    bits = x0 ^ x1

    float_bits = (bits >> u32(9)) | u32(0x3F800000)
    f = jax.lax.bitcast_convert_type(float_bits, jnp.float32)
    f = f - jnp.float32(1.0)
    mn = jnp.float32(1e-9)
    u = jnp.maximum(mn, f + mn)
    return jnp.log(-jnp.log(u))


def _make_body(blk, seq_len, vocab, edim):

    def _vq_body(keys_ref, z_ref, w0_ref, b0_ref, e0_ref,
                 w1_ref, b1_ref, e1_ref, out_ref, loss_ref):
        @pl.when(pl.program_id(0) == 0)
        def _init():
            loss_ref[0, 0] = jnp.float32(0.0)

        i = pl.program_id(0)
        t0 = i * blk
        b = t0 // seq_len
        l0 = t0 % seq_len
        base = (b * (vocab * seq_len) + l0).astype(jnp.uint32)
        row = jax.lax.broadcasted_iota(jnp.uint32, (blk, vocab), 0)
        col = jax.lax.broadcasted_iota(jnp.uint32, (blk, vocab), 1)
        cnt = base + row + col * np.uint32(seq_len)

        z = z_ref[...]
        log_v = jnp.log(jnp.float32(vocab))
        acc = jnp.float32(0.0)
        for idx, (w_ref, b_ref, e_ref) in enumerate(
                ((w0_ref, b0_ref, e0_ref), (w1_ref, b1_ref, e1_ref))):
            logits = jnp.dot(z, w_ref[...],
                             preferred_element_type=jnp.float32)
            logits = logits + b_ref[...]
            ll = _threefry_loglog(keys_ref[idx, 0].astype(jnp.uint32),
                                  keys_ref[idx, 1].astype(jnp.uint32), cnt)
            y = logits - ll
            if _TAU != 1.0:
                y = y * (1.0 / _TAU)
            y = y - jnp.max(y, axis=1, keepdims=True)
            ey = jnp.exp(y)
            soft = ey * (1.0 / jnp.sum(ey, axis=1, keepdims=True))
            out_ref[:, idx * edim:(idx + 1) * edim] = jnp.dot(
                soft.astype(jnp.bfloat16), e_ref[...],
                preferred_element_type=jnp.float32)
            x = logits - jnp.max(logits, axis=1, keepdims=True)
            ex = jnp.exp(x)
            zden = jnp.sum(ex, axis=1, keepdims=True)
            s1 = jnp.sum(ex * x, axis=1, keepdims=True)
            acc = acc + jnp.sum(s1 / zden + (log_v - jnp.log(zden)))
        loss_ref[0, 0] += acc

    return _vq_body


def kernel(seq, proj_w0, proj_b0, embed0, proj_w1, proj_b1, embed1):
    b, l, c = seq.shape
    v = proj_w0.shape[0]
    d = embed0.shape[1]
    tok = b * l

    z = seq.reshape(tok, c).astype(jnp.bfloat16)

    blk = 512
    while l % blk:
        blk //= 2
    grid = tok // blk
    out, loss = pl.pallas_call(
        _make_body(blk, l, v, d),
        grid=(grid,),
        in_specs=[
            pl.BlockSpec(memory_space=pltpu.SMEM),
            pl.BlockSpec((blk, c), lambda i: (i, 0)),
            pl.BlockSpec((c, v), lambda i: (0, 0)),
            pl.BlockSpec((1, v), lambda i: (0, 0)),
            pl.BlockSpec((v, d), lambda i: (0, 0)),
            pl.BlockSpec((c, v), lambda i: (0, 0)),
            pl.BlockSpec((1, v), lambda i: (0, 0)),
            pl.BlockSpec((v, d), lambda i: (0, 0)),
        ],
        out_specs=[
            pl.BlockSpec((blk, 2 * d), lambda i: (i, 0)),
            pl.BlockSpec((1, 1), lambda i: (0, 0),
                         memory_space=pltpu.SMEM),
        ],
        out_shape=[
            jax.ShapeDtypeStruct((tok, 2 * d), jnp.float32),
            jax.ShapeDtypeStruct((1, 1), jnp.float32),
        ],
    )(jnp.asarray(_KEYS), z,
      proj_w0.T.astype(jnp.bfloat16), proj_b0.reshape(1, v),
      embed0.astype(jnp.bfloat16),
      proj_w1.T.astype(jnp.bfloat16), proj_b1.reshape(1, v),
      embed1.astype(jnp.bfloat16))
    return out.reshape(b, l, 2 * d), loss[0, 0] / tok
